# baseline (speedup 1.0000x reference)
import jax
import jax.numpy as jnp
from jax.experimental import pallas as pl
from jax.experimental.pallas import tpu as pltpu

_S = 16
_H = 128
_E = 64
_NEMO = 32
_BR = 512


def _fused_body(idx_ref, emo_ref, states_ref, embed_ref, wih_ref, whh_ref,
                bih_ref, bhh_ref, out_ref):
    idx = idx_ref[...]
    emo = emo_ref[...]

    safe = jnp.where(emo >= 0, emo, _NEMO)
    cols = jax.lax.broadcasted_iota(jnp.int32, (1, _NEMO + 1), 1)
    onehot = (safe == cols).astype(jnp.float32)
    emb = jnp.dot(onehot, embed_ref[...],
                  preferred_element_type=jnp.float32)

    h_old = jnp.zeros((_BR, _H), jnp.float32)
    for s in range(_S):
        h_old = h_old + jnp.where(idx == s, states_ref[:, s, :], 0.0)

    gi = jnp.dot(emb, wih_ref[...],
                 preferred_element_type=jnp.float32) + bih_ref[...]
    gh = jnp.dot(h_old, whh_ref[...],
                 preferred_element_type=jnp.float32) + bhh_ref[...]
    r = jax.nn.sigmoid(gi[:, :_H] + gh[:, :_H])
    z = jax.nn.sigmoid(gi[:, _H:2 * _H] + gh[:, _H:2 * _H])
    n = jnp.tanh(gi[:, 2 * _H:] + r * gh[:, 2 * _H:])
    h_new = (1.0 - z) * n + z * h_old

    for s in range(_S):
        out_ref[:, s, :] = jnp.where(idx == s, h_new, states_ref[:, s, :])


def kernel(states, speaker_ids, emotion_ids, embed, W_ih, W_hh, b_ih, b_hh):
    B, S, H = states.shape
    nb = B // _BR
    idx = jnp.minimum(speaker_ids.astype(jnp.int32), S - 1).reshape(B, 1)
    emo = emotion_ids.astype(jnp.int32).reshape(B, 1)

    grid_spec = pl.GridSpec(
        grid=(nb,),
        in_specs=[
            pl.BlockSpec((_BR, 1), lambda i: (i, 0)),
            pl.BlockSpec((_BR, 1), lambda i: (i, 0)),
            pl.BlockSpec((_BR, S, H), lambda i: (i, 0, 0)),
            pl.BlockSpec((_NEMO + 1, _E), lambda i: (0, 0)),
            pl.BlockSpec((_E, 3 * _H), lambda i: (0, 0)),
            pl.BlockSpec((_H, 3 * _H), lambda i: (0, 0)),
            pl.BlockSpec((1, 3 * _H), lambda i: (0, 0)),
            pl.BlockSpec((1, 3 * _H), lambda i: (0, 0)),
        ],
        out_specs=pl.BlockSpec((_BR, S, H), lambda i: (i, 0, 0)),
    )
    return pl.pallas_call(
        _fused_body,
        grid_spec=grid_spec,
        out_shape=jax.ShapeDtypeStruct((B, S, H), states.dtype),
        compiler_params=pltpu.CompilerParams(
            dimension_semantics=("arbitrary",),
        ),
    )(idx, emo, states, embed, W_ih.T, W_hh.T,
      b_ih.reshape(1, -1), b_hh.reshape(1, -1))


# speedup vs baseline: 1.9029x; 1.9029x over previous
import functools

import jax
import jax.numpy as jnp
from jax import lax
from jax.experimental import pallas as pl
from jax.experimental.pallas import tpu as pltpu
from jax.experimental.pallas import tpu_sc as plsc

_S = 16
_H = 128
_E = 64
_NEMO = 32
_BR = 512
_NW = 32
_CH = 128


def _make_sc_gather(B):
    b_per_w = B // _NW
    nch = b_per_w // _CH
    mesh = plsc.VectorSubcoreMesh(core_axis_name="c", subcore_axis_name="s")

    @functools.partial(
        pl.kernel,
        out_type=jax.ShapeDtypeStruct((B, _H), jnp.float32),
        mesh=mesh,
        scratch_types=[
            pltpu.VMEM((_CH,), jnp.int32),
            pltpu.VMEM((_CH, _H), jnp.float32),
            pltpu.SemaphoreType.DMA,
        ],
    )
    def gather(states_hbm, flat_hbm, out_hbm, idx_v, rows_v, sem):
        wid = lax.axis_index("s") * 2 + lax.axis_index("c")
        base = wid * b_per_w
        for j in range(nch):
            off = base + j * _CH
            pltpu.sync_copy(flat_hbm.at[pl.ds(off, _CH)], idx_v)
            pltpu.async_copy(states_hbm.at[idx_v], rows_v, sem).wait()
            pltpu.sync_copy(rows_v, out_hbm.at[pl.ds(off, _CH)])

    return gather


def _make_sc_scatter(B):
    b_per_w = B // _NW
    nch = b_per_w // _CH
    mesh = plsc.VectorSubcoreMesh(core_axis_name="c", subcore_axis_name="s")

    @functools.partial(
        pl.kernel,
        out_type=(),
        mesh=mesh,
        scratch_types=[
            pltpu.VMEM((_CH,), jnp.int32),
            pltpu.VMEM((_CH, _H), jnp.float32),
            pltpu.SemaphoreType.DMA,
        ],
    )
    def scatter(out_ref, flat_hbm, hnew_hbm, idx_v, rows_v, sem):
        wid = lax.axis_index("s") * 2 + lax.axis_index("c")
        base = wid * b_per_w
        for j in range(nch):
            off = base + j * _CH
            pltpu.sync_copy(flat_hbm.at[pl.ds(off, _CH)], idx_v)
            pltpu.sync_copy(hnew_hbm.at[pl.ds(off, _CH)], rows_v)
            pltpu.async_copy(rows_v, out_ref.at[idx_v], sem).wait()

    return scatter


def _copy_gru_body(emo_ref, hold_ref, states_ref, embed_ref, wih_ref,
                   whh_ref, bih_ref, bhh_ref, out_ref, hnew_ref):
    out_ref[...] = states_ref[...]

    emo = emo_ref[...]
    h_old = hold_ref[...]
    safe = jnp.where(emo >= 0, emo, _NEMO)
    cols = lax.broadcasted_iota(jnp.int32, (1, _NEMO + 1), 1)
    onehot = (safe == cols).astype(jnp.float32)
    emb = jnp.dot(onehot, embed_ref[...],
                  preferred_element_type=jnp.float32)
    gi = jnp.dot(emb, wih_ref[...],
                 preferred_element_type=jnp.float32) + bih_ref[...]
    gh = jnp.dot(h_old, whh_ref[...],
                 preferred_element_type=jnp.float32) + bhh_ref[...]
    r = jax.nn.sigmoid(gi[:, :_H] + gh[:, :_H])
    z = jax.nn.sigmoid(gi[:, _H:2 * _H] + gh[:, _H:2 * _H])
    n = jnp.tanh(gi[:, 2 * _H:] + r * gh[:, 2 * _H:])
    hnew_ref[...] = (1.0 - z) * n + z * h_old


def _tc_copy_gru(states_flat, h_old, emo, embed, wih_t, whh_t, bih, bhh):
    BS, H = states_flat.shape
    B = BS // _S
    nb = B // _BR
    rows = _BR * _S
    return pl.pallas_call(
        _copy_gru_body,
        grid=(nb,),
        in_specs=[
            pl.BlockSpec((_BR, 1), lambda i: (i, 0)),
            pl.BlockSpec((_BR, _H), lambda i: (i, 0)),
            pl.BlockSpec((rows, _H), lambda i: (i, 0)),
            pl.BlockSpec((_NEMO + 1, _E), lambda i: (0, 0)),
            pl.BlockSpec((_E, 3 * _H), lambda i: (0, 0)),
            pl.BlockSpec((_H, 3 * _H), lambda i: (0, 0)),
            pl.BlockSpec((1, 3 * _H), lambda i: (0, 0)),
            pl.BlockSpec((1, 3 * _H), lambda i: (0, 0)),
        ],
        out_specs=[
            pl.BlockSpec((rows, _H), lambda i: (i, 0)),
            pl.BlockSpec((_BR, _H), lambda i: (i, 0)),
        ],
        out_shape=[
            jax.ShapeDtypeStruct((BS, H), jnp.float32),
            jax.ShapeDtypeStruct((B, _H), jnp.float32),
        ],
        compiler_params=pltpu.CompilerParams(
            dimension_semantics=("arbitrary",),
        ),
    )(emo, h_old, states_flat, embed, wih_t, whh_t, bih, bhh)


def kernel(states, speaker_ids, emotion_ids, embed, W_ih, W_hh, b_ih, b_hh):
    B, S, H = states.shape
    idx = jnp.minimum(speaker_ids.astype(jnp.int32), S - 1)
    flat = jnp.arange(B, dtype=jnp.int32) * S + idx
    emo = emotion_ids.astype(jnp.int32).reshape(B, 1)
    states_flat = states.reshape(B * S, H)

    h_old = _make_sc_gather(B)(states_flat, flat)
    out0, h_new = _tc_copy_gru(states_flat, h_old, emo, embed,
                               W_ih.T, W_hh.T,
                               b_ih.reshape(1, -1), b_hh.reshape(1, -1))
    out_ref = jax.new_ref(out0)
    _make_sc_scatter(B)(out_ref, flat, h_new)
    return out_ref[...].reshape(B, S, H)
